# baseline (speedup 1.0000x reference)
import functools

import jax
import jax.numpy as jnp
from jax import lax
from jax.experimental import pallas as pl
from jax.experimental.pallas import tpu as pltpu
from jax.experimental.pallas import tpu_sc as plsc

_L = 16
_VTW = 128
_VT = 4
_CH = _VT * _VTW
_SUB = 128
_NB = 3
_NC = 2
_NS = 16


def kernel(feat, in_idx):
    n_in, d = feat.shape
    n_out = in_idx.shape[0]
    ct_n = d // 8
    vt_n = (n_out + _VTW - 1) // _VTW
    n_pad = vt_n * _VTW

    idx32 = jnp.pad(in_idx.astype(jnp.int32), (0, n_pad - n_out),
                    constant_values=-1)

    nw = _NC * _NS
    n_chunks = (vt_n + _VT - 1) // _VT
    iters = (n_chunks + nw - 1) // nw
    last_vt0 = vt_n - _VT
    spread_mask = (1 << (n_in.bit_length() - 1)) - 1

    mesh = plsc.VectorSubcoreMesh(
        core_axis_name="c", subcore_axis_name="s",
        num_cores=_NC, num_subcores=_NS,
    )

    @functools.partial(
        pl.kernel,
        out_type=jax.ShapeDtypeStruct((ct_n, vt_n * 8 * _VTW), jnp.float32),
        mesh=mesh,
        scratch_types=[
            pltpu.VMEM((_NB, _CH), jnp.int32),
            pltpu.VMEM((_NB, _CH), jnp.int32),
            pltpu.VMEM((_NB, _CH), jnp.float32),
            pltpu.VMEM((_NB, _CH, d), jnp.float32),
            pltpu.VMEM((_NB, ct_n, _VT * 8 * _VTW), jnp.float32),
            pltpu.SemaphoreType.DMA((_NB,)),
            pltpu.SemaphoreType.DMA((_NB,)),
            pltpu.SemaphoreType.DMA((_NB,)),
        ],
        compiler_params=pltpu.CompilerParams(
            needs_layout_passes=False,
            use_tc_tiling_on_sc=False,
        ),
    )
    def body(feat_hbm, idx_hbm, out_hbm, idx_v, sidx_v, mval_v, rows_v, tr_v,
             isem, gsem, osem):
        wid = lax.axis_index("s") * _NC + lax.axis_index("c")
        lane = lax.iota(jnp.int32, _L)

        def vt0_of(c):
            return jnp.minimum(c * _VT, last_vt0)

        def step(i, carry):
            chunk_a = i * nw + wid

            @pl.when(jnp.logical_and(i < iters, chunk_a < n_chunks))
            def _():
                s = lax.rem(i, _NB)
                pltpu.async_copy(
                    idx_hbm.at[pl.ds(vt0_of(chunk_a) * _VTW, _CH)],
                    idx_v.at[s], isem.at[s])

            ib = i - 1
            chunk_b = ib * nw + wid

            @pl.when(jnp.logical_and(
                jnp.logical_and(ib >= 0, ib < iters), chunk_b < n_chunks))
            def _():
                s = lax.rem(ib, _NB)
                pltpu.make_async_copy(
                    idx_hbm.at[pl.ds(0, _CH)], idx_v.at[s],
                    isem.at[s]).wait()
                base = vt0_of(chunk_b) * _VTW

                def remap(g, c):
                    v = idx_v[s, pl.ds(g * _L, _L)]
                    valid = v >= 0
                    spread = (lane + (base + g * _L)) & spread_mask
                    sidx_v[s, pl.ds(g * _L, _L)] = jnp.where(valid, v, spread)
                    mval_v[s, pl.ds(g * _L, _L)] = jnp.where(
                        valid, jnp.float32(1.0), jnp.float32(0.0))
                    return c

                lax.fori_loop(0, _CH // _L, remap, 0)

                io = i - 4
                chunk_o = io * nw + wid

                @pl.when(jnp.logical_and(io >= 0, chunk_o < n_chunks))
                def _():
                    pltpu.make_async_copy(
                        tr_v.at[s],
                        out_hbm.at[pl.ds(0, ct_n), pl.ds(0, _VT * 8 * _VTW)],
                        osem.at[s]).wait()

                for j in range(0, _CH, _SUB):
                    pltpu.async_copy(
                        feat_hbm.at[sidx_v.at[s, pl.ds(j, _SUB)]],
                        rows_v.at[s, pl.ds(j, _SUB)],
                        gsem.at[s])

            ic = i - 2
            chunk_c = ic * nw + wid

            @pl.when(jnp.logical_and(
                jnp.logical_and(ic >= 0, ic < iters), chunk_c < n_chunks))
            def _():
                s = lax.rem(ic, _NB)
                pltpu.make_async_copy(
                    feat_hbm.at[pl.ds(0, _CH)], rows_v.at[s],
                    gsem.at[s]).wait()

                @plsc.parallel_loop(0, _CH // _L, 1, unroll=2)
                def trans(g):
                    mv = mval_v[s, pl.ds(g * _L, _L)]
                    rows = lane + g * _L
                    fl0 = lane + (g % (_VTW // _L)) * _L + (
                        g // (_VTW // _L)) * (8 * _VTW)
                    for h in range(d // _L):
                        for cc in range(_L):
                            colv = ((lane + cc) & (_L - 1)) + h * _L
                            x = plsc.load_gather(rows_v.at[s], [rows, colv])
                            flatv = ((colv & 7) << 7) + fl0
                            plsc.store_scatter(
                                tr_v.at[s], [colv >> 3, flatv], x * mv)

                vt0 = vt0_of(chunk_c)
                for ct in range(ct_n):
                    pltpu.async_copy(
                        tr_v.at[s, ct],
                        out_hbm.at[ct, pl.ds(vt0 * (8 * _VTW), _VT * 8 * _VTW)],
                        osem.at[s])

            return carry

        lax.fori_loop(0, iters + 2, step, 0)

        def drain(i, carry):
            it = iters - 1 - i
            chunk_t = it * nw + wid

            @pl.when(jnp.logical_and(it >= 0, chunk_t < n_chunks))
            def _():
                s = lax.rem(it, _NB)
                pltpu.make_async_copy(
                    tr_v.at[s],
                    out_hbm.at[pl.ds(0, ct_n), pl.ds(0, _VT * 8 * _VTW)],
                    osem.at[s]).wait()

            return carry

        lax.fori_loop(0, min(3, iters), drain, 0)

    phys = body(feat, idx32)
    phys = phys.reshape(ct_n, vt_n, 8, _VTW)
    out = phys.transpose(1, 3, 0, 2).reshape(n_pad, d)
    return out[:n_out]


# speedup vs baseline: 1.0133x; 1.0133x over previous
import functools

import jax
import jax.numpy as jnp
from jax import lax
from jax.experimental import pallas as pl
from jax.experimental.pallas import tpu as pltpu
from jax.experimental.pallas import tpu_sc as plsc

_L = 16
_VTW = 128
_VT = 4
_CH = _VT * _VTW
_SUB = 128
_NB = 3
_NC = 2
_NS = 16


def kernel(feat, in_idx):
    n_in, d = feat.shape
    n_out = in_idx.shape[0]
    ct_n = d // 8
    vt_n = (n_out + _VTW - 1) // _VTW
    n_pad = vt_n * _VTW

    idx32 = jnp.pad(in_idx.astype(jnp.int32), (0, n_pad - n_out),
                    constant_values=-1)

    nw = _NC * _NS
    n_chunks = (vt_n + _VT - 1) // _VT
    iters = (n_chunks + nw - 1) // nw
    last_vt0 = vt_n - _VT
    spread_mask = (1 << (n_in.bit_length() - 1)) - 1

    mesh = plsc.VectorSubcoreMesh(
        core_axis_name="c", subcore_axis_name="s",
        num_cores=_NC, num_subcores=_NS,
    )

    @functools.partial(
        pl.kernel,
        out_type=jax.ShapeDtypeStruct((ct_n, vt_n * 8 * _VTW), jnp.float32),
        mesh=mesh,
        scratch_types=[
            pltpu.VMEM((_NB, _CH), jnp.int32),
            pltpu.VMEM((_NB, _CH), jnp.int32),
            pltpu.VMEM((_NB, _CH), jnp.float32),
            pltpu.VMEM((_NB, _CH, d), jnp.float32),
            pltpu.VMEM((_NB, ct_n, _VT * 8 * _VTW), jnp.float32),
            pltpu.SemaphoreType.DMA((_NB,)),
            pltpu.SemaphoreType.DMA((_NB,)),
            pltpu.SemaphoreType.DMA((_NB,)),
        ],
        compiler_params=pltpu.CompilerParams(
            needs_layout_passes=False,
            use_tc_tiling_on_sc=False,
        ),
    )
    def body(feat_hbm, idx_hbm, out_hbm, idx_v, sidx_v, mval_v, rows_v, tr_v,
             isem, gsem, osem):
        wid = lax.axis_index("s") * _NC + lax.axis_index("c")
        lane = lax.iota(jnp.int32, _L)

        def vt0_of(c):
            return jnp.minimum(c * _VT, last_vt0)

        def step(i, carry):
            chunk_a = i * nw + wid

            @pl.when(jnp.logical_and(i < iters, chunk_a < n_chunks))
            def _():
                s = lax.rem(i, _NB)
                pltpu.async_copy(
                    idx_hbm.at[pl.ds(vt0_of(chunk_a) * _VTW, _CH)],
                    idx_v.at[s], isem.at[s])

            ib = i - 1
            chunk_b = ib * nw + wid

            @pl.when(jnp.logical_and(
                jnp.logical_and(ib >= 0, ib < iters), chunk_b < n_chunks))
            def _():
                s = lax.rem(ib, _NB)
                pltpu.make_async_copy(
                    idx_hbm.at[pl.ds(0, _CH)], idx_v.at[s],
                    isem.at[s]).wait()
                base = vt0_of(chunk_b) * _VTW

                @plsc.parallel_loop(0, _CH // _L, 1, unroll=2)
                def remap(g):
                    v = idx_v[s, pl.ds(g * _L, _L)]
                    valid = v >= 0
                    spread = (lane + (base + g * _L)) & spread_mask
                    sidx_v[s, pl.ds(g * _L, _L)] = jnp.where(valid, v, spread)
                    mval_v[s, pl.ds(g * _L, _L)] = jnp.where(
                        valid, jnp.float32(1.0), jnp.float32(0.0))

                io = i - 4
                chunk_o = io * nw + wid

                @pl.when(jnp.logical_and(io >= 0, chunk_o < n_chunks))
                def _():
                    pltpu.make_async_copy(
                        tr_v.at[s],
                        out_hbm.at[pl.ds(0, ct_n), pl.ds(0, _VT * 8 * _VTW)],
                        osem.at[s]).wait()

                for j in range(0, _CH, _SUB):
                    pltpu.async_copy(
                        feat_hbm.at[sidx_v.at[s, pl.ds(j, _SUB)]],
                        rows_v.at[s, pl.ds(j, _SUB)],
                        gsem.at[s])

            ic = i - 2
            chunk_c = ic * nw + wid

            @pl.when(jnp.logical_and(
                jnp.logical_and(ic >= 0, ic < iters), chunk_c < n_chunks))
            def _():
                s = lax.rem(ic, _NB)
                pltpu.make_async_copy(
                    feat_hbm.at[pl.ds(0, _CH)], rows_v.at[s],
                    gsem.at[s]).wait()

                @plsc.parallel_loop(0, _CH // _L, 1, unroll=2)
                def trans(g):
                    mv = mval_v[s, pl.ds(g * _L, _L)]
                    rows = lane + g * _L
                    fl0 = lane + (g % (_VTW // _L)) * _L + (
                        g // (_VTW // _L)) * (8 * _VTW)
                    for h in range(d // _L):
                        for cc in range(_L):
                            colv = ((lane + cc) & (_L - 1)) + h * _L
                            x = plsc.load_gather(rows_v.at[s], [rows, colv])
                            flatv = ((colv & 7) << 7) + fl0
                            plsc.store_scatter(
                                tr_v.at[s], [colv >> 3, flatv], x * mv)

                vt0 = vt0_of(chunk_c)
                for ct in range(ct_n):
                    pltpu.async_copy(
                        tr_v.at[s, ct],
                        out_hbm.at[ct, pl.ds(vt0 * (8 * _VTW), _VT * 8 * _VTW)],
                        osem.at[s])

            return carry

        lax.fori_loop(0, iters + 2, step, 0)

        def drain(i, carry):
            it = iters - 1 - i
            chunk_t = it * nw + wid

            @pl.when(jnp.logical_and(it >= 0, chunk_t < n_chunks))
            def _():
                s = lax.rem(it, _NB)
                pltpu.make_async_copy(
                    tr_v.at[s],
                    out_hbm.at[pl.ds(0, ct_n), pl.ds(0, _VT * 8 * _VTW)],
                    osem.at[s]).wait()

            return carry

        lax.fori_loop(0, min(3, iters), drain, 0)

    phys = body(feat, idx32)
    phys = phys.reshape(ct_n, vt_n, 8, _VTW)
    out = phys.transpose(1, 3, 0, 2).reshape(n_pad, d)
    return out[:n_out]


# speedup vs baseline: 1.5795x; 1.5588x over previous
import functools

import jax
import jax.numpy as jnp
from jax import lax
from jax.experimental import pallas as pl
from jax.experimental.pallas import tpu as pltpu
from jax.experimental.pallas import tpu_sc as plsc

_L = 16
_VTW = 128
_VT = 4
_CH = _VT * _VTW
_SUB = 128
_NB = 3
_NC = 2
_NS = 16


def _feat_to_rowmajor(feat):
    n_in, d = feat.shape
    featT = feat.T
    units = n_in // _VTW
    tail = n_in - units * _VTW
    lin_rows = n_in * d // _VTW
    tail128 = feat[units * _VTW:].reshape(tail * d // _VTW, _VTW)

    nw = _NC * _NS
    iters = (units + nw - 1) // nw

    mesh = plsc.VectorSubcoreMesh(
        core_axis_name="c", subcore_axis_name="s",
        num_cores=_NC, num_subcores=_NS,
    )

    @functools.partial(
        pl.kernel,
        out_type=jax.ShapeDtypeStruct((lin_rows, _VTW), jnp.float32),
        mesh=mesh,
        scratch_types=[
            pltpu.VMEM((2, d, _VTW), jnp.float32),
            pltpu.VMEM((2, d, _VTW), jnp.float32),
            pltpu.SemaphoreType.DMA((2,)),
            pltpu.SemaphoreType.DMA((2,)),
        ],
        compiler_params=pltpu.CompilerParams(
            needs_layout_passes=False,
            use_tc_tiling_on_sc=True,
        ),
    )
    def conv(ft_hbm, tail_hbm, lin_hbm, blk_v, tblk_v, isem, osem):
        wid = lax.axis_index("s") * _NC + lax.axis_index("c")
        lane = lax.iota(jnp.int32, _L)

        @pl.when(wid == 0)
        def _():
            pltpu.sync_copy(tail_hbm, lin_hbm.at[pl.ds(units * d, tail * d
                                                       // _VTW)])

        def step(i, carry):
            ua = i * nw + wid

            @pl.when(jnp.logical_and(i < iters, ua < units))
            def _():
                s = lax.rem(i, 2)
                pltpu.async_copy(
                    ft_hbm.at[pl.ds(0, d), pl.ds(ua * _VTW, _VTW)],
                    blk_v.at[s], isem.at[s])

            ub = (i - 1) * nw + wid

            @pl.when(jnp.logical_and(
                jnp.logical_and(i >= 1, i - 1 < iters), ub < units))
            def _():
                s = lax.rem(i - 1, 2)
                pltpu.make_async_copy(
                    ft_hbm.at[pl.ds(0, d), pl.ds(0, _VTW)], blk_v.at[s],
                    isem.at[s]).wait()

                uo = (i - 3) * nw + wid

                @pl.when(jnp.logical_and(i >= 3, uo < units))
                def _():
                    pltpu.make_async_copy(
                        tblk_v.at[s], lin_hbm.at[pl.ds(0, d)],
                        osem.at[s]).wait()

                lane32 = lane * d

                @plsc.parallel_loop(0, _VTW // _L, 1, unroll=2)
                def trans(gv):
                    vl = lane + gv * _L
                    fl0 = lane32 + gv * (_L * d)
                    for cc in range(d):
                        cvec = (lane + cc) & (d - 1)
                        x = plsc.load_gather(blk_v.at[s], [cvec, vl])
                        flat = fl0 + cvec
                        plsc.store_scatter(
                            tblk_v.at[s],
                            [flat >> 7, flat & (_VTW - 1)], x)

                pltpu.async_copy(tblk_v.at[s],
                                 lin_hbm.at[pl.ds(ub * d, d)],
                                 osem.at[s])

            return carry

        lax.fori_loop(0, iters + 1, step, 0)

        def drain(i, carry):
            ut = (iters - 1 - i) * nw + wid

            @pl.when(jnp.logical_and(iters - 1 - i >= 0, ut < units))
            def _():
                s = lax.rem(iters - 1 - i, 2)
                pltpu.make_async_copy(
                    tblk_v.at[s], lin_hbm.at[pl.ds(0, d)],
                    osem.at[s]).wait()

            return carry

        lax.fori_loop(0, min(2, iters), drain, 0)

    return conv(featT, tail128).reshape(n_in, d)


def kernel(feat, in_idx):
    n_in, d = feat.shape
    n_out = in_idx.shape[0]
    feat = _feat_to_rowmajor(feat)
    ct_n = d // 8
    vt_n = (n_out + _VTW - 1) // _VTW
    n_pad = vt_n * _VTW

    idx32 = jnp.pad(in_idx.astype(jnp.int32), (0, n_pad - n_out),
                    constant_values=-1)

    nw = _NC * _NS
    n_chunks = (vt_n + _VT - 1) // _VT
    iters = (n_chunks + nw - 1) // nw
    last_vt0 = vt_n - _VT
    spread_mask = (1 << (n_in.bit_length() - 1)) - 1

    mesh = plsc.VectorSubcoreMesh(
        core_axis_name="c", subcore_axis_name="s",
        num_cores=_NC, num_subcores=_NS,
    )

    @functools.partial(
        pl.kernel,
        out_type=jax.ShapeDtypeStruct((ct_n, vt_n * 8 * _VTW), jnp.float32),
        mesh=mesh,
        scratch_types=[
            pltpu.VMEM((_NB, _CH), jnp.int32),
            pltpu.VMEM((_NB, _CH), jnp.int32),
            pltpu.VMEM((_NB, _CH), jnp.float32),
            pltpu.VMEM((_NB, _CH, d), jnp.float32),
            pltpu.VMEM((_NB, ct_n, _VT * 8 * _VTW), jnp.float32),
            pltpu.SemaphoreType.DMA((_NB,)),
            pltpu.SemaphoreType.DMA((_NB,)),
            pltpu.SemaphoreType.DMA((_NB,)),
        ],
        compiler_params=pltpu.CompilerParams(
            needs_layout_passes=False,
            use_tc_tiling_on_sc=False,
        ),
    )
    def body(feat_hbm, idx_hbm, out_hbm, idx_v, sidx_v, mval_v, rows_v, tr_v,
             isem, gsem, osem):
        wid = lax.axis_index("s") * _NC + lax.axis_index("c")
        lane = lax.iota(jnp.int32, _L)

        def vt0_of(c):
            return jnp.minimum(c * _VT, last_vt0)

        def step(i, carry):
            chunk_a = i * nw + wid

            @pl.when(jnp.logical_and(i < iters, chunk_a < n_chunks))
            def _():
                s = lax.rem(i, _NB)
                pltpu.async_copy(
                    idx_hbm.at[pl.ds(vt0_of(chunk_a) * _VTW, _CH)],
                    idx_v.at[s], isem.at[s])

            ib = i - 1
            chunk_b = ib * nw + wid

            @pl.when(jnp.logical_and(
                jnp.logical_and(ib >= 0, ib < iters), chunk_b < n_chunks))
            def _():
                s = lax.rem(ib, _NB)
                pltpu.make_async_copy(
                    idx_hbm.at[pl.ds(0, _CH)], idx_v.at[s],
                    isem.at[s]).wait()
                base = vt0_of(chunk_b) * _VTW

                @plsc.parallel_loop(0, _CH // _L, 1, unroll=2)
                def remap(g):
                    v = idx_v[s, pl.ds(g * _L, _L)]
                    valid = v >= 0
                    spread = (lane + (base + g * _L)) & spread_mask
                    sidx_v[s, pl.ds(g * _L, _L)] = jnp.where(valid, v, spread)
                    mval_v[s, pl.ds(g * _L, _L)] = jnp.where(
                        valid, jnp.float32(1.0), jnp.float32(0.0))

                io = i - 4
                chunk_o = io * nw + wid

                @pl.when(jnp.logical_and(io >= 0, chunk_o < n_chunks))
                def _():
                    pltpu.make_async_copy(
                        tr_v.at[s],
                        out_hbm.at[pl.ds(0, ct_n), pl.ds(0, _VT * 8 * _VTW)],
                        osem.at[s]).wait()

                for j in range(0, _CH, _SUB):
                    pltpu.async_copy(
                        feat_hbm.at[sidx_v.at[s, pl.ds(j, _SUB)]],
                        rows_v.at[s, pl.ds(j, _SUB)],
                        gsem.at[s])

            ic = i - 2
            chunk_c = ic * nw + wid

            @pl.when(jnp.logical_and(
                jnp.logical_and(ic >= 0, ic < iters), chunk_c < n_chunks))
            def _():
                s = lax.rem(ic, _NB)
                pltpu.make_async_copy(
                    feat_hbm.at[pl.ds(0, _CH)], rows_v.at[s],
                    gsem.at[s]).wait()

                @plsc.parallel_loop(0, _CH // _L, 1, unroll=2)
                def trans(g):
                    mv = mval_v[s, pl.ds(g * _L, _L)]
                    rows = lane + g * _L
                    fl0 = lane + (g % (_VTW // _L)) * _L + (
                        g // (_VTW // _L)) * (8 * _VTW)
                    for h in range(d // _L):
                        for cc in range(_L):
                            colv = ((lane + cc) & (_L - 1)) + h * _L
                            x = plsc.load_gather(rows_v.at[s], [rows, colv])
                            flatv = ((colv & 7) << 7) + fl0
                            plsc.store_scatter(
                                tr_v.at[s], [colv >> 3, flatv], x * mv)

                vt0 = vt0_of(chunk_c)
                for ct in range(ct_n):
                    pltpu.async_copy(
                        tr_v.at[s, ct],
                        out_hbm.at[ct, pl.ds(vt0 * (8 * _VTW), _VT * 8 * _VTW)],
                        osem.at[s])

            return carry

        lax.fori_loop(0, iters + 2, step, 0)

        def drain(i, carry):
            it = iters - 1 - i
            chunk_t = it * nw + wid

            @pl.when(jnp.logical_and(it >= 0, chunk_t < n_chunks))
            def _():
                s = lax.rem(it, _NB)
                pltpu.make_async_copy(
                    tr_v.at[s],
                    out_hbm.at[pl.ds(0, ct_n), pl.ds(0, _VT * 8 * _VTW)],
                    osem.at[s]).wait()

            return carry

        lax.fori_loop(0, min(3, iters), drain, 0)

    phys = body(feat, idx32)
    phys = phys.reshape(ct_n, vt_n, 8, _VTW)
    out = phys.transpose(1, 3, 0, 2).reshape(n_pad, d)
    return out[:n_out]


# speedup vs baseline: 1.9632x; 1.2429x over previous
import functools

import jax
import jax.numpy as jnp
from jax import lax
from jax.experimental import pallas as pl
from jax.experimental.pallas import tpu as pltpu
from jax.experimental.pallas import tpu_sc as plsc

_L = 16
_VTW = 128
_VT = 4
_CH = _VT * _VTW
_SUB = 128
_NB = 3
_NC = 2
_NS = 16


def _feat_to_rowmajor(feat):
    n_in, d = feat.shape
    featT = feat.T
    cw = 2 * _VTW
    units = n_in // cw
    tail = n_in - units * cw
    lin_rows = n_in * d // _VTW
    rpu = cw * d // _VTW
    tail128 = feat[units * cw:].reshape(tail * d // _VTW, _VTW)

    nw = _NC * _NS
    iters = (units + nw - 1) // nw

    mesh = plsc.VectorSubcoreMesh(
        core_axis_name="c", subcore_axis_name="s",
        num_cores=_NC, num_subcores=_NS,
    )

    @functools.partial(
        pl.kernel,
        out_type=jax.ShapeDtypeStruct((lin_rows, _VTW), jnp.float32),
        mesh=mesh,
        scratch_types=[
            pltpu.VMEM((2, d, cw), jnp.float32),
            pltpu.VMEM((2, cw * d // _VTW, _VTW), jnp.float32),
            pltpu.SemaphoreType.DMA((2,)),
            pltpu.SemaphoreType.DMA((2,)),
        ],
        compiler_params=pltpu.CompilerParams(
            needs_layout_passes=False,
            use_tc_tiling_on_sc=True,
        ),
    )
    def conv(ft_hbm, tail_hbm, lin_hbm, blk_v, tblk_v, isem, osem):
        wid = lax.axis_index("s") * _NC + lax.axis_index("c")
        lane = lax.iota(jnp.int32, _L)

        @pl.when(wid == 0)
        def _():
            pltpu.sync_copy(tail_hbm, lin_hbm.at[pl.ds(units * rpu, tail * d
                                                       // _VTW)])

        def step(i, carry):
            ua = i * nw + wid

            @pl.when(jnp.logical_and(i < iters, ua < units))
            def _():
                s = lax.rem(i, 2)
                pltpu.async_copy(
                    ft_hbm.at[pl.ds(0, d), pl.ds(ua * cw, cw)],
                    blk_v.at[s], isem.at[s])

            ub = (i - 1) * nw + wid

            @pl.when(jnp.logical_and(
                jnp.logical_and(i >= 1, i - 1 < iters), ub < units))
            def _():
                s = lax.rem(i - 1, 2)
                pltpu.make_async_copy(
                    ft_hbm.at[pl.ds(0, d), pl.ds(0, cw)], blk_v.at[s],
                    isem.at[s]).wait()

                uo = (i - 3) * nw + wid

                @pl.when(jnp.logical_and(i >= 3, uo < units))
                def _():
                    pltpu.make_async_copy(
                        tblk_v.at[s], lin_hbm.at[pl.ds(0, rpu)],
                        osem.at[s]).wait()

                lane32 = lane * d

                @plsc.parallel_loop(0, cw // _L, 1, unroll=2)
                def trans(gv):
                    vl = lane + gv * _L
                    fl0 = lane32 + gv * (_L * d)
                    for cc in range(d):
                        cvec = (lane + cc) & (d - 1)
                        x = plsc.load_gather(blk_v.at[s], [cvec, vl])
                        flat = fl0 + cvec
                        plsc.store_scatter(
                            tblk_v.at[s],
                            [flat >> 7, flat & (_VTW - 1)], x)

                pltpu.async_copy(tblk_v.at[s],
                                 lin_hbm.at[pl.ds(ub * rpu, rpu)],
                                 osem.at[s])

            return carry

        lax.fori_loop(0, iters + 1, step, 0)

        def drain(i, carry):
            ut = (iters - 1 - i) * nw + wid

            @pl.when(jnp.logical_and(iters - 1 - i >= 0, ut < units))
            def _():
                s = lax.rem(iters - 1 - i, 2)
                pltpu.make_async_copy(
                    tblk_v.at[s], lin_hbm.at[pl.ds(0, rpu)],
                    osem.at[s]).wait()

            return carry

        lax.fori_loop(0, min(2, iters), drain, 0)

    return conv(featT, tail128).reshape(n_in, d)


def kernel(feat, in_idx):
    n_in, d = feat.shape
    n_out = in_idx.shape[0]
    feat = _feat_to_rowmajor(feat)
    ct_n = d // 8
    vt_n = (n_out + _VTW - 1) // _VTW
    n_pad = vt_n * _VTW

    idx32 = jnp.pad(in_idx.astype(jnp.int32), (0, n_pad - n_out),
                    constant_values=-1)

    nw = _NC * _NS
    n_chunks = (vt_n + _VT - 1) // _VT
    iters = (n_chunks + nw - 1) // nw
    last_vt0 = vt_n - _VT
    spread_mask = (1 << (n_in.bit_length() - 1)) - 1

    mesh = plsc.VectorSubcoreMesh(
        core_axis_name="c", subcore_axis_name="s",
        num_cores=_NC, num_subcores=_NS,
    )

    @functools.partial(
        pl.kernel,
        out_type=jax.ShapeDtypeStruct((ct_n, vt_n * 8 * _VTW), jnp.float32),
        mesh=mesh,
        scratch_types=[
            pltpu.VMEM((_NB, _CH), jnp.int32),
            pltpu.VMEM((_NB, _CH), jnp.int32),
            pltpu.VMEM((_NB, _CH), jnp.float32),
            pltpu.VMEM((_NB, _CH, d), jnp.float32),
            pltpu.VMEM((_NB, ct_n, _VT * 8 * _VTW), jnp.float32),
            pltpu.SemaphoreType.DMA((_NB,)),
            pltpu.SemaphoreType.DMA((_NB,)),
            pltpu.SemaphoreType.DMA((_NB,)),
        ],
        compiler_params=pltpu.CompilerParams(
            needs_layout_passes=False,
            use_tc_tiling_on_sc=False,
        ),
    )
    def body(feat_hbm, idx_hbm, out_hbm, idx_v, sidx_v, mval_v, rows_v, tr_v,
             isem, gsem, osem):
        wid = lax.axis_index("s") * _NC + lax.axis_index("c")
        lane = lax.iota(jnp.int32, _L)

        def vt0_of(c):
            return jnp.minimum(c * _VT, last_vt0)

        def step(i, carry):
            chunk_a = i * nw + wid

            @pl.when(jnp.logical_and(i < iters, chunk_a < n_chunks))
            def _():
                s = lax.rem(i, _NB)
                pltpu.async_copy(
                    idx_hbm.at[pl.ds(vt0_of(chunk_a) * _VTW, _CH)],
                    idx_v.at[s], isem.at[s])

            ib = i - 1
            chunk_b = ib * nw + wid

            @pl.when(jnp.logical_and(
                jnp.logical_and(ib >= 0, ib < iters), chunk_b < n_chunks))
            def _():
                s = lax.rem(ib, _NB)
                pltpu.make_async_copy(
                    idx_hbm.at[pl.ds(0, _CH)], idx_v.at[s],
                    isem.at[s]).wait()
                base = vt0_of(chunk_b) * _VTW

                @plsc.parallel_loop(0, _CH // _L, 1, unroll=2)
                def remap(g):
                    v = idx_v[s, pl.ds(g * _L, _L)]
                    valid = v >= 0
                    spread = (lane + (base + g * _L)) & spread_mask
                    sidx_v[s, pl.ds(g * _L, _L)] = jnp.where(valid, v, spread)
                    mval_v[s, pl.ds(g * _L, _L)] = jnp.where(
                        valid, jnp.float32(1.0), jnp.float32(0.0))

                io = i - 4
                chunk_o = io * nw + wid

                @pl.when(jnp.logical_and(io >= 0, chunk_o < n_chunks))
                def _():
                    pltpu.make_async_copy(
                        tr_v.at[s],
                        out_hbm.at[pl.ds(0, ct_n), pl.ds(0, _VT * 8 * _VTW)],
                        osem.at[s]).wait()

                for j in range(0, _CH, _SUB):
                    pltpu.async_copy(
                        feat_hbm.at[sidx_v.at[s, pl.ds(j, _SUB)]],
                        rows_v.at[s, pl.ds(j, _SUB)],
                        gsem.at[s])

            ic = i - 2
            chunk_c = ic * nw + wid

            @pl.when(jnp.logical_and(
                jnp.logical_and(ic >= 0, ic < iters), chunk_c < n_chunks))
            def _():
                s = lax.rem(ic, _NB)
                pltpu.make_async_copy(
                    feat_hbm.at[pl.ds(0, _CH)], rows_v.at[s],
                    gsem.at[s]).wait()

                @plsc.parallel_loop(0, _CH // _L, 1, unroll=2)
                def trans(g):
                    mv = mval_v[s, pl.ds(g * _L, _L)]
                    rows = lane + g * _L
                    fl0 = lane + (g % (_VTW // _L)) * _L + (
                        g // (_VTW // _L)) * (8 * _VTW)
                    for h in range(d // _L):
                        for cc in range(_L):
                            colv = ((lane + cc) & (_L - 1)) + h * _L
                            x = plsc.load_gather(rows_v.at[s], [rows, colv])
                            flatv = ((colv & 7) << 7) + fl0
                            plsc.store_scatter(
                                tr_v.at[s], [colv >> 3, flatv], x * mv)

                vt0 = vt0_of(chunk_c)
                for ct in range(ct_n):
                    pltpu.async_copy(
                        tr_v.at[s, ct],
                        out_hbm.at[ct, pl.ds(vt0 * (8 * _VTW), _VT * 8 * _VTW)],
                        osem.at[s])

            return carry

        lax.fori_loop(0, iters + 2, step, 0)

        def drain(i, carry):
            it = iters - 1 - i
            chunk_t = it * nw + wid

            @pl.when(jnp.logical_and(it >= 0, chunk_t < n_chunks))
            def _():
                s = lax.rem(it, _NB)
                pltpu.make_async_copy(
                    tr_v.at[s],
                    out_hbm.at[pl.ds(0, ct_n), pl.ds(0, _VT * 8 * _VTW)],
                    osem.at[s]).wait()

            return carry

        lax.fori_loop(0, min(3, iters), drain, 0)

    phys = body(feat, idx32)
    phys = phys.reshape(ct_n, vt_n, 8, _VTW)
    out = phys.transpose(1, 3, 0, 2).reshape(n_pad, d)
    return out[:n_out]
